# speedup vs baseline: 1.0395x; 1.0395x over previous
import functools

import jax
import jax.numpy as jnp
from jax.experimental import pallas as pl
from jax.experimental.pallas import tpu as pltpu

_MIB = 1024 * 1024
_TKM = 384


def _round_up(a: int, b: int) -> int:
    return (a + b - 1) // b * b


def _glu_mlp_kernel(x_ref, wgv_ref, wo_ref, o_ref, *, tkm):
    m = pl.program_id(1)

    wgv_b = wgv_ref[...].astype(jnp.bfloat16)
    fused = jnp.dot(x_ref[...], wgv_b, preferred_element_type=jnp.float32)
    gate = fused[:, :tkm]
    value = fused[:, tkm:]
    h = (gate * jax.nn.sigmoid(gate) * value).astype(jnp.bfloat16)
    wo_b = wo_ref[...].astype(jnp.bfloat16)
    part = jnp.dot(h, wo_b, preferred_element_type=jnp.float32)

    @pl.when(m == 0)
    def _init():
        o_ref[...] = part

    @pl.when(m > 0)
    def _accum():
        o_ref[...] += part


@jax.jit
def kernel(x, wgv, wo):
    H = x.shape[-1]
    lead_shape = x.shape[:-1]
    m_pad = wo.shape[0]
    tkm = _TKM
    n_m = m_pad // tkm

    x2d = x.reshape(-1, H).astype(jnp.bfloat16)
    N = x2d.shape[0]

    tm = min(1024, max(128, _round_up(N, 128)))
    n_pad = _round_up(N, tm)
    if n_pad != N:
        x2d = jnp.pad(x2d, ((0, n_pad - N), (0, 0)))
    n_tiles = n_pad // tm

    cost = pl.CostEstimate(
        flops=6 * N * H * m_pad,
        transcendentals=N * m_pad,
        bytes_accessed=(2 * N * H * 4) + 3 * H * m_pad * 4 * n_tiles // 2,
    )

    out2d = pl.pallas_call(
        functools.partial(_glu_mlp_kernel, tkm=tkm),
        out_shape=jax.ShapeDtypeStruct((n_pad, H), jnp.float32),
        grid_spec=pltpu.PrefetchScalarGridSpec(
            num_scalar_prefetch=0,
            grid=(n_tiles, n_m),
            in_specs=[
                pl.BlockSpec((tm, H), lambda i, m: (i, 0)),
                pl.BlockSpec((H, 2 * tkm), lambda i, m: (0, m)),
                pl.BlockSpec((tkm, H), lambda i, m: (m, 0)),
            ],
            out_specs=pl.BlockSpec((tm, H), lambda i, m: (i, 0)),
        ),
        compiler_params=pltpu.CompilerParams(
            dimension_semantics=("parallel", "arbitrary"),
            vmem_limit_bytes=60 * _MIB,
        ),
        cost_estimate=cost,
    )(x2d, wgv, wo)

    if n_pad != N:
        out2d = out2d[:N]
    return out2d.reshape(*lead_shape, H)


# speedup vs baseline: 1.0678x; 1.0272x over previous
import functools

import jax
import jax.numpy as jnp
from jax.experimental import pallas as pl
from jax.experimental.pallas import tpu as pltpu

_MIB = 1024 * 1024
_TKM = 384


def _round_up(a: int, b: int) -> int:
    return (a + b - 1) // b * b


def _glu_mlp_kernel(x_ref, wgv_ref, wo_ref, o_ref, *, tkm):
    m = pl.program_id(1)

    fused = jnp.dot(x_ref[...], wgv_ref[...], preferred_element_type=jnp.float32)
    gate = fused[:, :tkm]
    value = fused[:, tkm:]
    h = gate * jax.nn.sigmoid(gate) * value
    part = jnp.dot(h, wo_ref[...], preferred_element_type=jnp.float32)

    @pl.when(m == 0)
    def _init():
        o_ref[...] = part

    @pl.when(m > 0)
    def _accum():
        o_ref[...] += part


@jax.jit
def kernel(x, wgv, wo):
    H = x.shape[-1]
    lead_shape = x.shape[:-1]
    m_pad = wo.shape[0]
    tkm = _TKM
    n_m = m_pad // tkm

    x2d = x.reshape(-1, H)
    N = x2d.shape[0]

    tm = min(1024, max(128, _round_up(N, 128)))
    n_pad = _round_up(N, tm)
    if n_pad != N:
        x2d = jnp.pad(x2d, ((0, n_pad - N), (0, 0)))
    n_tiles = n_pad // tm

    cost = pl.CostEstimate(
        flops=6 * N * H * m_pad,
        transcendentals=N * m_pad,
        bytes_accessed=(2 * N * H * 4) + 3 * H * m_pad * 4 * n_tiles // 2,
    )

    out2d = pl.pallas_call(
        functools.partial(_glu_mlp_kernel, tkm=tkm),
        out_shape=jax.ShapeDtypeStruct((n_pad, H), jnp.float32),
        grid_spec=pltpu.PrefetchScalarGridSpec(
            num_scalar_prefetch=0,
            grid=(n_tiles, n_m),
            in_specs=[
                pl.BlockSpec((tm, H), lambda i, m: (i, 0)),
                pl.BlockSpec((H, 2 * tkm), lambda i, m: (0, m)),
                pl.BlockSpec((tkm, H), lambda i, m: (m, 0)),
            ],
            out_specs=pl.BlockSpec((tm, H), lambda i, m: (i, 0)),
        ),
        compiler_params=pltpu.CompilerParams(
            dimension_semantics=("parallel", "arbitrary"),
            vmem_limit_bytes=60 * _MIB,
        ),
        cost_estimate=cost,
    )(x2d, wgv, wo)

    if n_pad != N:
        out2d = out2d[:N]
    return out2d.reshape(*lead_shape, H)


# speedup vs baseline: 1.1724x; 1.0979x over previous
import functools

import jax
import jax.numpy as jnp
from jax.experimental import pallas as pl
from jax.experimental.pallas import tpu as pltpu

_MIB = 1024 * 1024
_TKM = 384


def _round_up(a: int, b: int) -> int:
    return (a + b - 1) // b * b


def _glu_mlp_kernel(x_ref, wgv_ref, wo_ref, o_ref, *, tkm):
    m = pl.program_id(1)

    fused = jnp.dot(x_ref[...], wgv_ref[...], preferred_element_type=jnp.float32)
    gate = fused[:, :tkm]
    value = fused[:, tkm:]
    h = gate * jax.nn.sigmoid(gate) * value

    H = o_ref.shape[1]
    n_chunks = 4
    cw = H // n_chunks
    for c in range(n_chunks):
        sl = slice(c * cw, (c + 1) * cw)
        part = jnp.dot(h, wo_ref[:, sl], preferred_element_type=jnp.float32)
        prev = jnp.where(m == 0, 0.0, o_ref[:, sl])
        o_ref[:, sl] = part + prev


@jax.jit
def kernel(x, wgv, wo):
    H = x.shape[-1]
    lead_shape = x.shape[:-1]
    m_pad = wo.shape[0]
    tkm = _TKM
    n_m = m_pad // tkm

    x2d = x.reshape(-1, H)
    N = x2d.shape[0]

    tm = min(1024, max(128, _round_up(N, 128)))
    n_pad = _round_up(N, tm)
    if n_pad != N:
        x2d = jnp.pad(x2d, ((0, n_pad - N), (0, 0)))
    n_tiles = n_pad // tm

    cost = pl.CostEstimate(
        flops=6 * N * H * m_pad,
        transcendentals=N * m_pad,
        bytes_accessed=(2 * N * H * 4) + 3 * H * m_pad * 4 * n_tiles // 2,
    )

    out2d = pl.pallas_call(
        functools.partial(_glu_mlp_kernel, tkm=tkm),
        out_shape=jax.ShapeDtypeStruct((n_pad, H), jnp.float32),
        grid_spec=pltpu.PrefetchScalarGridSpec(
            num_scalar_prefetch=0,
            grid=(n_tiles, n_m),
            in_specs=[
                pl.BlockSpec((tm, H), lambda i, m: (i, 0)),
                pl.BlockSpec((H, 2 * tkm), lambda i, m: (0, m)),
                pl.BlockSpec((tkm, H), lambda i, m: (m, 0)),
            ],
            out_specs=pl.BlockSpec((tm, H), lambda i, m: (i, 0)),
        ),
        compiler_params=pltpu.CompilerParams(
            dimension_semantics=("parallel", "arbitrary"),
            vmem_limit_bytes=60 * _MIB,
        ),
        cost_estimate=cost,
    )(x2d, wgv, wo)

    if n_pad != N:
        out2d = out2d[:N]
    return out2d.reshape(*lead_shape, H)
